# Initial kernel scaffold; baseline (speedup 1.0000x reference)
import jax
import jax.numpy as jnp
from jax.experimental import pallas as pl


def kernel(R, Rij, idx_i, idx_j):
    raise NotImplementedError("write your pallas kernel here")


# speedup vs baseline: 3.3108x; 3.3108x over previous
import functools

import jax
import jax.numpy as jnp
from jax import lax
from jax.experimental import pallas as pl
from jax.experimental.pallas import tpu as pltpu
from jax.experimental.pallas import tpu_sc as plsc

N_NODES = 100000
N_EDGES = 6400000

NC = 2
NS = 16
NW = NC * NS
L = 16

GROUP = 128
CHUNK_E = 2048
CHUNK_W = 6 * CHUNK_E
N_DMA = CHUNK_W // GROUP
N_CHUNKS = N_EDGES // CHUNK_E
ACC_W_SUB = 18768
ACC_W = ACC_W_SUB * NS


def _sc_body(rij_hbm, idxi_hbm, idxj_hbm, out0_hbm, out1_hbm,
             rij_v, ii_v, jj_v, data_v, idxe_v, acc):
    cid = lax.axis_index("c")
    sid = lax.axis_index("s")
    wid = sid * NC + cid

    iota = lax.iota(jnp.int32, L)
    zf = jnp.zeros((L,), jnp.float32)

    def _z(k, carry):
        data_v[pl.ds(k * L, L)] = zf
        return carry
    lax.fori_loop(0, CHUNK_W // L, _z, 0)
    a0 = sid * ACC_W_SUB
    pltpu.sync_copy(data_v, acc.at[pl.ds(a0, CHUNK_W)])
    pltpu.sync_copy(data_v.at[pl.ds(0, ACC_W_SUB - CHUNK_W)],
                    acc.at[pl.ds(a0 + CHUNK_W, ACC_W_SUB - CHUNK_W)])
    plsc.subcore_barrier()

    nloc = (N_CHUNKS - wid + NW - 1) // NW

    def _chunk(i, carry):
        c = wid + i * NW
        pltpu.sync_copy(rij_hbm.at[pl.ds(c * (3 * CHUNK_E), 3 * CHUNK_E)], rij_v)
        pltpu.sync_copy(idxi_hbm.at[pl.ds(c * CHUNK_E, CHUNK_E)], ii_v)
        pltpu.sync_copy(idxj_hbm.at[pl.ds(c * CHUNK_E, CHUNK_E)], jj_v)

        def _grp(k, carry2):
            e0 = k * L
            ii3 = ii_v[pl.ds(e0, L)] * 3
            jj3 = jj_v[pl.ds(e0, L)] * 3
            base = 3 * e0 + 3 * iota
            x = plsc.load_gather(rij_v, [base])
            y = plsc.load_gather(rij_v, [base + 1])
            z = plsc.load_gather(rij_v, [base + 2])
            g = -2.0 * jnp.exp(-(x * x + y * y + z * z))
            row0 = k // 8
            col = (e0 % GROUP) + iota
            vals = (g * x, g * y, g * z)
            for s in range(3):
                data_v[pl.ds(s * CHUNK_E + e0, L)] = vals[s]
                data_v[pl.ds((s + 3) * CHUNK_E + e0, L)] = -vals[s]
                plsc.store_scatter(
                    idxe_v, [jnp.full((L,), s * (CHUNK_E // GROUP) + row0,
                                      jnp.int32), col], ii3 + s)
                plsc.store_scatter(
                    idxe_v, [jnp.full((L,), (s + 3) * (CHUNK_E // GROUP) + row0,
                                      jnp.int32), col], jj3 + s)
            return carry2
        lax.fori_loop(0, CHUNK_E // L, _grp, 0)

        def _scat(j, carry2):
            pltpu.sync_copy(data_v.at[pl.ds(j * GROUP, GROUP)],
                            acc.at[idxe_v.at[j]], add=True)
            return carry2
        lax.fori_loop(0, N_DMA, _scat, 0)
        return carry

    lax.fori_loop(0, nloc, _chunk, 0)

    plsc.subcore_barrier()

    rem = ACC_W_SUB - CHUNK_W

    pltpu.sync_copy(acc.at[pl.ds(a0, CHUNK_W)], data_v)

    @pl.when(cid == 0)
    def _():
        pltpu.sync_copy(data_v, out0_hbm.at[pl.ds(a0, CHUNK_W)])

    @pl.when(cid == 1)
    def _():
        pltpu.sync_copy(data_v, out1_hbm.at[pl.ds(a0, CHUNK_W)])

    pltpu.sync_copy(acc.at[pl.ds(a0 + CHUNK_W, rem)], data_v.at[pl.ds(0, rem)])

    @pl.when(cid == 0)
    def _():
        pltpu.sync_copy(data_v.at[pl.ds(0, rem)],
                        out0_hbm.at[pl.ds(a0 + CHUNK_W, rem)])

    @pl.when(cid == 1)
    def _():
        pltpu.sync_copy(data_v.at[pl.ds(0, rem)],
                        out1_hbm.at[pl.ds(a0 + CHUNK_W, rem)])


_sc_scatter = functools.partial(
    pl.kernel,
    out_type=(jax.ShapeDtypeStruct((ACC_W,), jnp.float32),
              jax.ShapeDtypeStruct((ACC_W,), jnp.float32)),
    mesh=plsc.VectorSubcoreMesh(core_axis_name="c", subcore_axis_name="s"),
    compiler_params=pltpu.CompilerParams(needs_layout_passes=False),
    scratch_types=[
        pltpu.VMEM((3 * CHUNK_E,), jnp.float32),
        pltpu.VMEM((CHUNK_E,), jnp.int32),
        pltpu.VMEM((CHUNK_E,), jnp.int32),
        pltpu.VMEM((CHUNK_W,), jnp.float32),
        pltpu.VMEM((N_DMA, GROUP), jnp.int32),
        pltpu.VMEM_SHARED((ACC_W,), jnp.float32),
    ],
)(_sc_body)


def _combine_body(a_ref, b_ref, o_ref):
    o_ref[...] = a_ref[...] + b_ref[...]


_R128 = ACC_W // 128
_combine = pl.pallas_call(
    _combine_body,
    out_shape=jax.ShapeDtypeStruct((_R128, 128), jnp.float32),
)


@jax.jit
def kernel(R, Rij, idx_i, idx_j):
    rij_flat = Rij.reshape(3 * N_EDGES)
    idxi = idx_i.astype(jnp.int32)
    idxj = idx_j.astype(jnp.int32)
    p0, p1 = _sc_scatter(rij_flat, idxi, idxj)
    summed = _combine(p0.reshape(_R128, 128), p1.reshape(_R128, 128))
    return summed.reshape(ACC_W // 3, 3)[:N_NODES]


# speedup vs baseline: 3.5264x; 1.0651x over previous
import functools

import jax
import jax.numpy as jnp
from jax import lax
from jax.experimental import pallas as pl
from jax.experimental.pallas import tpu as pltpu
from jax.experimental.pallas import tpu_sc as plsc

N_NODES = 100000
N_EDGES = 6400000

NC = 2
NS = 16
NW = NC * NS
L = 16

GROUP = 128
CHUNK_E = 2048
CHUNK_W = 6 * CHUNK_E
N_DMA = CHUNK_W // GROUP
N_CHUNKS = N_EDGES // CHUNK_E
ACC_W_SUB = 18768
ACC_W = ACC_W_SUB * NS


def _sc_body(rij_hbm, idxi_hbm, idxj_hbm, out0_hbm, out1_hbm,
             rij_v, ii_v, jj_v, data_v, idxe_v, acc, sem):
    cid = lax.axis_index("c")
    sid = lax.axis_index("s")
    wid = sid * NC + cid

    iota = lax.iota(jnp.int32, L)
    zf = jnp.zeros((L,), jnp.float32)

    def _z(k, carry):
        data_v[pl.ds(k * L, L)] = zf
        return carry
    lax.fori_loop(0, CHUNK_W // L, _z, 0)
    a0 = sid * ACC_W_SUB
    pltpu.sync_copy(data_v, acc.at[pl.ds(a0, CHUNK_W)])
    pltpu.sync_copy(data_v.at[pl.ds(0, ACC_W_SUB - CHUNK_W)],
                    acc.at[pl.ds(a0 + CHUNK_W, ACC_W_SUB - CHUNK_W)])
    plsc.subcore_barrier()

    nloc = (N_CHUNKS - wid + NW - 1) // NW

    def _chunk(i, carry):
        c = wid + i * NW
        pltpu.sync_copy(rij_hbm.at[pl.ds(c * (3 * CHUNK_E), 3 * CHUNK_E)], rij_v)
        pltpu.sync_copy(idxi_hbm.at[pl.ds(c * CHUNK_E, CHUNK_E)], ii_v)
        pltpu.sync_copy(idxj_hbm.at[pl.ds(c * CHUNK_E, CHUNK_E)], jj_v)

        def _grp(k, carry2):
            e0 = k * L
            ii3 = ii_v[pl.ds(e0, L)] * 3
            jj3 = jj_v[pl.ds(e0, L)] * 3
            base = 3 * e0 + 3 * iota
            x = plsc.load_gather(rij_v, [base])
            y = plsc.load_gather(rij_v, [base + 1])
            z = plsc.load_gather(rij_v, [base + 2])
            g = -2.0 * jnp.exp(-(x * x + y * y + z * z))
            row0 = k // 8
            col = (e0 % GROUP) + iota
            vals = (g * x, g * y, g * z)
            for s in range(3):
                data_v[pl.ds(s * CHUNK_E + e0, L)] = vals[s]
                data_v[pl.ds((s + 3) * CHUNK_E + e0, L)] = -vals[s]
                plsc.store_scatter(
                    idxe_v, [jnp.full((L,), s * (CHUNK_E // GROUP) + row0,
                                      jnp.int32), col], ii3 + s)
                plsc.store_scatter(
                    idxe_v, [jnp.full((L,), (s + 3) * (CHUNK_E // GROUP) + row0,
                                      jnp.int32), col], jj3 + s)
            return carry2
        lax.fori_loop(0, CHUNK_E // L, _grp, 0)

        def _scat(j, carry2):
            pltpu.async_copy(data_v.at[pl.ds(j * GROUP, GROUP)],
                             acc.at[idxe_v.at[j]], sem, add=True)
            return carry2
        lax.fori_loop(0, N_DMA, _scat, 0)
        pltpu.make_async_copy(data_v, acc.at[pl.ds(0, CHUNK_W)], sem).wait()
        return carry

    lax.fori_loop(0, nloc, _chunk, 0)

    plsc.subcore_barrier()

    rem = ACC_W_SUB - CHUNK_W

    pltpu.sync_copy(acc.at[pl.ds(a0, CHUNK_W)], data_v)

    @pl.when(cid == 0)
    def _():
        pltpu.sync_copy(data_v, out0_hbm.at[pl.ds(a0, CHUNK_W)])

    @pl.when(cid == 1)
    def _():
        pltpu.sync_copy(data_v, out1_hbm.at[pl.ds(a0, CHUNK_W)])

    pltpu.sync_copy(acc.at[pl.ds(a0 + CHUNK_W, rem)], data_v.at[pl.ds(0, rem)])

    @pl.when(cid == 0)
    def _():
        pltpu.sync_copy(data_v.at[pl.ds(0, rem)],
                        out0_hbm.at[pl.ds(a0 + CHUNK_W, rem)])

    @pl.when(cid == 1)
    def _():
        pltpu.sync_copy(data_v.at[pl.ds(0, rem)],
                        out1_hbm.at[pl.ds(a0 + CHUNK_W, rem)])


_sc_scatter = functools.partial(
    pl.kernel,
    out_type=(jax.ShapeDtypeStruct((ACC_W,), jnp.float32),
              jax.ShapeDtypeStruct((ACC_W,), jnp.float32)),
    mesh=plsc.VectorSubcoreMesh(core_axis_name="c", subcore_axis_name="s"),
    compiler_params=pltpu.CompilerParams(needs_layout_passes=False),
    scratch_types=[
        pltpu.VMEM((3 * CHUNK_E,), jnp.float32),
        pltpu.VMEM((CHUNK_E,), jnp.int32),
        pltpu.VMEM((CHUNK_E,), jnp.int32),
        pltpu.VMEM((CHUNK_W,), jnp.float32),
        pltpu.VMEM((N_DMA, GROUP), jnp.int32),
        pltpu.VMEM_SHARED((ACC_W,), jnp.float32),
        pltpu.SemaphoreType.DMA,
    ],
)(_sc_body)


def _combine_body(a_ref, b_ref, o_ref):
    o_ref[...] = a_ref[...] + b_ref[...]


_R128 = ACC_W // 128
_combine = pl.pallas_call(
    _combine_body,
    out_shape=jax.ShapeDtypeStruct((_R128, 128), jnp.float32),
)


@jax.jit
def kernel(R, Rij, idx_i, idx_j):
    rij_flat = Rij.reshape(3 * N_EDGES)
    idxi = idx_i.astype(jnp.int32)
    idxj = idx_j.astype(jnp.int32)
    p0, p1 = _sc_scatter(rij_flat, idxi, idxj)
    summed = _combine(p0.reshape(_R128, 128), p1.reshape(_R128, 128))
    return summed.reshape(ACC_W // 3, 3)[:N_NODES]


# speedup vs baseline: 29.6729x; 8.4146x over previous
import functools

import jax
import jax.numpy as jnp
from jax import lax
from jax.experimental import pallas as pl
from jax.experimental.pallas import tpu as pltpu
from jax.experimental.pallas import tpu_sc as plsc

N_NODES = 100000
N_EDGES = 6400000

NC = 2
NS = 16
NW = NC * NS
L = 16

GROUP = 128
CHUNK_E = 2048
CHUNK_W = 6 * CHUNK_E
N_DMA = CHUNK_W // GROUP
N_CHUNKS = N_EDGES // CHUNK_E
N_PAD = 100096
ACC_W = 3 * N_PAD
ACC_W_SUB = ACC_W // NS


def _sc_body(xs_hbm, ys_hbm, zs_hbm, idxi_hbm, idxj_hbm, out0_hbm, out1_hbm,
             xv, yv, zv, ii_v, jj_v, data_v, idxe_v, acc, sem):
    cid = lax.axis_index("c")
    sid = lax.axis_index("s")
    wid = sid * NC + cid

    iota = lax.iota(jnp.int32, L)
    zf = jnp.zeros((L,), jnp.float32)

    def _z(k, carry):
        data_v[pl.ds(k * L, L)] = zf
        return carry
    lax.fori_loop(0, CHUNK_W // L, _z, 0)
    a0 = sid * ACC_W_SUB
    pltpu.sync_copy(data_v, acc.at[pl.ds(a0, CHUNK_W)])
    pltpu.sync_copy(data_v.at[pl.ds(0, ACC_W_SUB - CHUNK_W)],
                    acc.at[pl.ds(a0 + CHUNK_W, ACC_W_SUB - CHUNK_W)])
    plsc.subcore_barrier()

    nloc = (N_CHUNKS - wid + NW - 1) // NW

    def _chunk(i, carry):
        c = wid + i * NW
        e0g = c * CHUNK_E
        pltpu.sync_copy(xs_hbm.at[pl.ds(e0g, CHUNK_E)], xv)
        pltpu.sync_copy(ys_hbm.at[pl.ds(e0g, CHUNK_E)], yv)
        pltpu.sync_copy(zs_hbm.at[pl.ds(e0g, CHUNK_E)], zv)
        pltpu.sync_copy(idxi_hbm.at[pl.ds(e0g, CHUNK_E)], ii_v)
        pltpu.sync_copy(idxj_hbm.at[pl.ds(e0g, CHUNK_E)], jj_v)

        def _grp(k, carry2):
            e0 = k * L
            ii = ii_v[pl.ds(e0, L)]
            jj = jj_v[pl.ds(e0, L)]
            x = xv[pl.ds(e0, L)]
            y = yv[pl.ds(e0, L)]
            z = zv[pl.ds(e0, L)]
            g = -2.0 * jnp.exp(-(x * x + y * y + z * z))
            row0 = k // 8
            col = (e0 % GROUP) + iota
            vals = (g * x, g * y, g * z)
            for s in range(3):
                data_v[pl.ds(s * CHUNK_E + e0, L)] = vals[s]
                data_v[pl.ds((s + 3) * CHUNK_E + e0, L)] = -vals[s]
                plsc.store_scatter(
                    idxe_v, [jnp.full((L,), s * (CHUNK_E // GROUP) + row0,
                                      jnp.int32), col], ii + s * N_PAD)
                plsc.store_scatter(
                    idxe_v, [jnp.full((L,), (s + 3) * (CHUNK_E // GROUP) + row0,
                                      jnp.int32), col], jj + s * N_PAD)
            return carry2
        lax.fori_loop(0, CHUNK_E // L, _grp, 0)

        def _scat(j, carry2):
            pltpu.async_copy(data_v.at[pl.ds(j * GROUP, GROUP)],
                             acc.at[idxe_v.at[j]], sem, add=True)
            return carry2
        lax.fori_loop(0, N_DMA, _scat, 0)
        pltpu.make_async_copy(data_v, acc.at[pl.ds(0, CHUNK_W)], sem).wait()
        return carry

    lax.fori_loop(0, nloc, _chunk, 0)

    plsc.subcore_barrier()

    rem = ACC_W_SUB - CHUNK_W

    pltpu.sync_copy(acc.at[pl.ds(a0, CHUNK_W)], data_v)

    @pl.when(cid == 0)
    def _():
        pltpu.sync_copy(data_v, out0_hbm.at[pl.ds(a0, CHUNK_W)])

    @pl.when(cid == 1)
    def _():
        pltpu.sync_copy(data_v, out1_hbm.at[pl.ds(a0, CHUNK_W)])

    pltpu.sync_copy(acc.at[pl.ds(a0 + CHUNK_W, rem)], data_v.at[pl.ds(0, rem)])

    @pl.when(cid == 0)
    def _():
        pltpu.sync_copy(data_v.at[pl.ds(0, rem)],
                        out0_hbm.at[pl.ds(a0 + CHUNK_W, rem)])

    @pl.when(cid == 1)
    def _():
        pltpu.sync_copy(data_v.at[pl.ds(0, rem)],
                        out1_hbm.at[pl.ds(a0 + CHUNK_W, rem)])


_sc_scatter = functools.partial(
    pl.kernel,
    out_type=(jax.ShapeDtypeStruct((ACC_W,), jnp.float32),
              jax.ShapeDtypeStruct((ACC_W,), jnp.float32)),
    mesh=plsc.VectorSubcoreMesh(core_axis_name="c", subcore_axis_name="s"),
    compiler_params=pltpu.CompilerParams(needs_layout_passes=False),
    scratch_types=[
        pltpu.VMEM((CHUNK_E,), jnp.float32),
        pltpu.VMEM((CHUNK_E,), jnp.float32),
        pltpu.VMEM((CHUNK_E,), jnp.float32),
        pltpu.VMEM((CHUNK_E,), jnp.int32),
        pltpu.VMEM((CHUNK_E,), jnp.int32),
        pltpu.VMEM((CHUNK_W,), jnp.float32),
        pltpu.VMEM((N_DMA, GROUP), jnp.int32),
        pltpu.VMEM_SHARED((ACC_W,), jnp.float32),
        pltpu.SemaphoreType.DMA,
    ],
)(_sc_body)


def _combine_body(a_ref, b_ref, o_ref):
    o_ref[...] = a_ref[...] + b_ref[...]


_R128 = ACC_W // 128
_combine = pl.pallas_call(
    _combine_body,
    out_shape=jax.ShapeDtypeStruct((_R128, 128), jnp.float32),
)


@jax.jit
def kernel(R, Rij, idx_i, idx_j):
    xs = Rij[:, 0]
    ys = Rij[:, 1]
    zs = Rij[:, 2]
    idxi = idx_i.astype(jnp.int32)
    idxj = idx_j.astype(jnp.int32)
    p0, p1 = _sc_scatter(xs, ys, zs, idxi, idxj)
    summed = _combine(p0.reshape(_R128, 128), p1.reshape(_R128, 128))
    planar = summed.reshape(3, N_PAD)
    return planar[:, :N_NODES].T


# speedup vs baseline: 45.4658x; 1.5322x over previous
import functools

import jax
import jax.numpy as jnp
from jax import lax
from jax.experimental import pallas as pl
from jax.experimental.pallas import tpu as pltpu
from jax.experimental.pallas import tpu_sc as plsc

N_NODES = 100000
N_EDGES = 6400000

NC = 2
NS = 16
NW = NC * NS
L = 16

GROUP = 128
CHUNK_E = 3200
CHUNK_W = 6 * CHUNK_E
N_DMA = CHUNK_W // GROUP
SEG_R = CHUNK_E // GROUP
N_CHUNKS = N_EDGES // CHUNK_E
N_PAD = 100096
ACC_W = 3 * N_PAD
ACC_W_SUB = ACC_W // NS


def _sc_body(xs_hbm, ys_hbm, zs_hbm, idxi_hbm, idxj_hbm, out0_hbm, out1_hbm,
             xa, ya, za, iia, jja, xb, yb, zb, iib, jjb,
             data_a, data_b, idxe_a, idxe_b, acc,
             sem_sa, sem_sb, sem_la, sem_lb):
    cid = lax.axis_index("c")
    sid = lax.axis_index("s")
    wid = sid * NC + cid

    iota = lax.iota(jnp.int32, L)
    zf = jnp.zeros((L,), jnp.float32)

    def _z(k, carry):
        data_a[pl.ds(k * L, L)] = zf
        return carry
    lax.fori_loop(0, CHUNK_W // L, _z, 0)
    a0 = sid * ACC_W_SUB
    pltpu.sync_copy(data_a.at[pl.ds(0, ACC_W_SUB)], acc.at[pl.ds(a0, ACC_W_SUB)])
    plsc.subcore_barrier()

    nloc = (N_CHUNKS - wid + NW - 1) // NW

    ins_a = (xa, ya, za, iia, jja)
    ins_b = (xb, yb, zb, iib, jjb)
    srcs = (xs_hbm, ys_hbm, zs_hbm, idxi_hbm, idxj_hbm)

    def _issue_loads(i, ins, sem):
        e0g = (wid + i * NW) * CHUNK_E
        for src, dst in zip(srcs, ins):
            pltpu.async_copy(src.at[pl.ds(e0g, CHUNK_E)], dst, sem)

    def _wait_loads(i, ins, sem):
        e0g = (wid + i * NW) * CHUNK_E
        for src, dst in zip(srcs, ins):
            pltpu.make_async_copy(src.at[pl.ds(e0g, CHUNK_E)], dst, sem).wait()

    def _compute(ins, data_v, idxe_v):
        xv, yv, zv, ii_v, jj_v = ins

        def _grp(k, carry2):
            e0 = k * L
            ii = ii_v[pl.ds(e0, L)]
            jj = jj_v[pl.ds(e0, L)]
            x = xv[pl.ds(e0, L)]
            y = yv[pl.ds(e0, L)]
            z = zv[pl.ds(e0, L)]
            g = -2.0 * jnp.exp(-(x * x + y * y + z * z))
            row0 = k // 8
            col = (e0 % GROUP) + iota
            vals = (g * x, g * y, g * z)
            for s in range(3):
                data_v[pl.ds(s * CHUNK_E + e0, L)] = vals[s]
                data_v[pl.ds((s + 3) * CHUNK_E + e0, L)] = -vals[s]
                plsc.store_scatter(
                    idxe_v, [jnp.full((L,), s * SEG_R + row0, jnp.int32), col],
                    ii + s * N_PAD)
                plsc.store_scatter(
                    idxe_v, [jnp.full((L,), (s + 3) * SEG_R + row0, jnp.int32),
                             col], jj + s * N_PAD)
            return carry2
        lax.fori_loop(0, CHUNK_E // L, _grp, 0)

    def _issue_scat(data_v, idxe_v, sem):
        def _scat(j, carry2):
            pltpu.async_copy(data_v.at[pl.ds(j * GROUP, GROUP)],
                             acc.at[idxe_v.at[j]], sem, add=True)
            return carry2
        lax.fori_loop(0, N_DMA, _scat, 0)

    def _drain_scat(data_v, sem):
        pltpu.make_async_copy(data_v, acc.at[pl.ds(0, CHUNK_W)], sem).wait()

    def _half(i, ins, sem_l, data_v, idxe_v, sem_s):
        _wait_loads(i, ins, sem_l)

        @pl.when(i >= 2)
        def _():
            _drain_scat(data_v, sem_s)

        _compute(ins, data_v, idxe_v)

        @pl.when(i + 2 < nloc)
        def _():
            _issue_loads(i + 2, ins, sem_l)

        _issue_scat(data_v, idxe_v, sem_s)

    _issue_loads(0, ins_a, sem_la)
    _issue_loads(1, ins_b, sem_lb)

    def _pair(ii2, carry):
        i0 = 2 * ii2
        _half(i0, ins_a, sem_la, data_a, idxe_a, sem_sa)

        @pl.when(i0 + 1 < nloc)
        def _():
            _half(i0 + 1, ins_b, sem_lb, data_b, idxe_b, sem_sb)

        return carry

    lax.fori_loop(0, (nloc + 1) // 2, _pair, 0)

    _drain_scat(data_a, sem_sa)
    _drain_scat(data_b, sem_sb)

    plsc.subcore_barrier()

    pltpu.sync_copy(acc.at[pl.ds(a0, ACC_W_SUB)], data_a.at[pl.ds(0, ACC_W_SUB)])

    @pl.when(cid == 0)
    def _():
        pltpu.sync_copy(data_a.at[pl.ds(0, ACC_W_SUB)],
                        out0_hbm.at[pl.ds(a0, ACC_W_SUB)])

    @pl.when(cid == 1)
    def _():
        pltpu.sync_copy(data_a.at[pl.ds(0, ACC_W_SUB)],
                        out1_hbm.at[pl.ds(a0, ACC_W_SUB)])


_sc_scatter = functools.partial(
    pl.kernel,
    out_type=(jax.ShapeDtypeStruct((ACC_W,), jnp.float32),
              jax.ShapeDtypeStruct((ACC_W,), jnp.float32)),
    mesh=plsc.VectorSubcoreMesh(core_axis_name="c", subcore_axis_name="s"),
    compiler_params=pltpu.CompilerParams(needs_layout_passes=False),
    scratch_types=[
        pltpu.VMEM((CHUNK_E,), jnp.float32),
        pltpu.VMEM((CHUNK_E,), jnp.float32),
        pltpu.VMEM((CHUNK_E,), jnp.float32),
        pltpu.VMEM((CHUNK_E,), jnp.int32),
        pltpu.VMEM((CHUNK_E,), jnp.int32),
        pltpu.VMEM((CHUNK_E,), jnp.float32),
        pltpu.VMEM((CHUNK_E,), jnp.float32),
        pltpu.VMEM((CHUNK_E,), jnp.float32),
        pltpu.VMEM((CHUNK_E,), jnp.int32),
        pltpu.VMEM((CHUNK_E,), jnp.int32),
        pltpu.VMEM((CHUNK_W,), jnp.float32),
        pltpu.VMEM((CHUNK_W,), jnp.float32),
        pltpu.VMEM((N_DMA, GROUP), jnp.int32),
        pltpu.VMEM((N_DMA, GROUP), jnp.int32),
        pltpu.VMEM_SHARED((ACC_W,), jnp.float32),
        pltpu.SemaphoreType.DMA,
        pltpu.SemaphoreType.DMA,
        pltpu.SemaphoreType.DMA,
        pltpu.SemaphoreType.DMA,
    ],
)(_sc_body)


def _combine_body(a_ref, b_ref, o_ref):
    o_ref[...] = a_ref[...] + b_ref[...]


_R128 = ACC_W // 128
_combine = pl.pallas_call(
    _combine_body,
    out_shape=jax.ShapeDtypeStruct((_R128, 128), jnp.float32),
)


@jax.jit
def kernel(R, Rij, idx_i, idx_j):
    xs = Rij[:, 0]
    ys = Rij[:, 1]
    zs = Rij[:, 2]
    idxi = idx_i.astype(jnp.int32)
    idxj = idx_j.astype(jnp.int32)
    p0, p1 = _sc_scatter(xs, ys, zs, idxi, idxj)
    summed = _combine(p0.reshape(_R128, 128), p1.reshape(_R128, 128))
    planar = summed.reshape(3, N_PAD)
    return planar[:, :N_NODES].T
